# Initial kernel scaffold; baseline (speedup 1.0000x reference)
import jax
import jax.numpy as jnp
from jax.experimental import pallas as pl


def kernel(edges, node_score, memorized_embedding, rel_emb, query_src_ts_emb, query_rel_emb, left_W, left_b, right_W, right_b, center_W, center_b, max_edges):
    raise NotImplementedError("write your pallas kernel here")


# speedup vs baseline: 1.0195x; 1.0195x over previous
import jax
import jax.numpy as jnp
from jax.experimental import pallas as pl


def kernel(edges, node_score, memorized_embedding, rel_emb, query_src_ts_emb, query_rel_emb, left_W, left_b, right_W, right_b, center_W, center_b, max_edges):
    num_nodes = memorized_embedding.shape[0]
    n = edges.shape[0]
    b = query_src_ts_emb.shape[0]
    e_per = n // b
    seg = edges[:, 6]
    dst = edges[:, 7]

    Whl, Wrl, Wsl, Wql = left_W[:, :128], left_W[:, 128:256], left_W[:, 256:384], left_W[:, 384:]
    Whr, Wrr, Wsr, Wqr = right_W[:, :128], right_W[:, 128:256], right_W[:, 256:384], right_W[:, 384:]

    QL = query_src_ts_emb @ Wsl.T + query_rel_emb @ Wql.T + left_b
    QR = query_src_ts_emb @ Wsr.T + query_rel_emb @ Wqr.T + right_b

    hvi = memorized_embedding[seg]
    hvj = memorized_embedding[dst]

    lp = hvi @ Whl.T + rel_emb @ Wrl.T + jnp.repeat(QL, e_per, axis=0)
    rp = hvj @ Whr.T + rel_emb @ Wrr.T + jnp.repeat(QR, e_per, axis=0)
    left = jnp.where(lp > 0, lp, 0.01 * lp)
    right = jnp.where(rp > 0, rp, 0.01 * rp)
    center = right @ center_W.T + center_b
    logits = jnp.sum(left * center, axis=-1)

    m = jax.ops.segment_max(logits, seg, num_segments=num_nodes)
    ex = jnp.exp(logits - m[seg])
    s = jax.ops.segment_sum(ex, seg, num_segments=num_nodes)
    target_att = (ex / s[seg]) * node_score[seg]

    ta = target_att.reshape(b, e_per)
    k_static = 64
    pruned_att, idx = jax.lax.top_k(ta, k_static)
    valid = jnp.arange(k_static) < max_edges
    pruned_att = jnp.where(valid[None, :], pruned_att, jnp.float32(0.0))
    orig_indices = (idx + jnp.arange(b, dtype=idx.dtype)[:, None] * e_per).reshape(-1)
    pruned_att_flat = pruned_att.reshape(-1)

    pruned_dst = jnp.take(dst, orig_indices, axis=0)
    updated_node_score = jnp.zeros((num_nodes,), dtype=pruned_att_flat.dtype).at[pruned_dst].add(pruned_att_flat)
    return (updated_node_score, pruned_att_flat, orig_indices)


# speedup vs baseline: 1.0711x; 1.0506x over previous
import functools

import jax
import jax.numpy as jnp
from jax.experimental import pallas as pl
from jax.experimental.pallas import tpu as pltpu

N = 160000
B = 64
E_PER = N // B
D = 128
K = 64


def _dense_body(gl_ref, gr_ref, rel_ref, qs_ref, qr_ref, lw_ref, rw_ref,
                cw_ref, lb_ref, rb_ref, cb_ref, out_ref):
    g = gl_ref[0]
    h = gr_ref[0]
    r = rel_ref[0]
    qs = qs_ref[0]
    qr = qr_ref[0]

    dn = (((1,), (1,)), ((), ()))
    f32 = jnp.float32

    def dot(a, b):
        return jax.lax.dot_general(a, b, dn, preferred_element_type=f32)

    ql = dot(qs, lw_ref[:, 256:384]) + dot(qr, lw_ref[:, 384:512]) + lb_ref[...]
    qright = dot(qs, rw_ref[:, 256:384]) + dot(qr, rw_ref[:, 384:512]) + rb_ref[...]

    lp = dot(g, lw_ref[:, 0:128]) + dot(r, lw_ref[:, 128:256]) + ql
    rp = dot(h, rw_ref[:, 0:128]) + dot(r, rw_ref[:, 128:256]) + qright
    left = jnp.where(lp > 0, lp, 0.01 * lp)
    right = jnp.where(rp > 0, rp, 0.01 * rp)
    center = dot(right, cw_ref[...]) + cb_ref[...]
    out_ref[0, 0, :] = jnp.sum(left * center, axis=1)


def _dense_logits(gl, gr, rel, qse, qre, lw, rw, cw, lb, rb, cb):
    full = lambda shape: pl.BlockSpec(shape, lambda q: (0,) * len(shape))
    per_q3 = pl.BlockSpec((1, E_PER, D), lambda q: (q, 0, 0))
    per_q1 = pl.BlockSpec((1, 1, D), lambda q: (q, 0, 0))
    out = pl.pallas_call(
        _dense_body,
        grid=(B,),
        in_specs=[per_q3, per_q3, per_q3, per_q1, per_q1,
                  full((2 * D, 4 * D)), full((2 * D, 4 * D)),
                  full((2 * D, 2 * D)),
                  full((1, 2 * D)), full((1, 2 * D)), full((1, 2 * D))],
        out_specs=pl.BlockSpec((1, 1, E_PER), lambda q: (q, 0, 0)),
        out_shape=jax.ShapeDtypeStruct((B, 1, E_PER), jnp.float32),
    )(gl, gr, rel, qse, qre, lw, rw, cw, lb.reshape(1, -1),
      rb.reshape(1, -1), cb.reshape(1, -1))
    return out.reshape(B, E_PER)


def _topk_body(me_ref, ta_ref, vals_ref, idx_ref, work_ref):
    work_ref[...] = ta_ref[...]
    col = jax.lax.broadcasted_iota(jnp.int32, (B, E_PER), 1)
    kcol = jax.lax.broadcasted_iota(jnp.int32, (B, K), 1)
    krow = jax.lax.broadcasted_iota(jnp.int32, (B, K), 0)

    def step(i, carry):
        vals, idxs = carry
        w = work_ref[...]
        m = jnp.max(w, axis=1)
        eq = w == m[:, None]
        amx = jnp.min(jnp.where(eq, col, E_PER), axis=1)
        vals = jnp.where(kcol == i, m[:, None], vals)
        idxs = jnp.where(kcol == i, amx[:, None], idxs)
        work_ref[...] = jnp.where(col == amx[:, None], jnp.float32(-1.0), w)
        return vals, idxs

    vals0 = jnp.zeros((B, K), jnp.float32)
    idxs0 = jnp.zeros((B, K), jnp.int32)
    vals, idxs = jax.lax.fori_loop(0, K, step, (vals0, idxs0))
    me = me_ref[0]
    vals_ref[...] = jnp.where(kcol < me, vals, jnp.float32(0.0))
    idx_ref[...] = idxs + krow * E_PER


def _topk(ta, max_edges):
    me = jnp.asarray(max_edges, jnp.int32).reshape(1)
    return pl.pallas_call(
        _topk_body,
        in_specs=[pl.BlockSpec(memory_space=pltpu.SMEM),
                  pl.BlockSpec((B, E_PER), lambda: (0, 0))],
        out_specs=[pl.BlockSpec((B, K), lambda: (0, 0)),
                   pl.BlockSpec((B, K), lambda: (0, 0))],
        out_shape=[jax.ShapeDtypeStruct((B, K), jnp.float32),
                   jax.ShapeDtypeStruct((B, K), jnp.int32)],
        scratch_shapes=[pltpu.VMEM((B, E_PER), jnp.float32)],
    )(me, ta)


def kernel(edges, node_score, memorized_embedding, rel_emb, query_src_ts_emb,
           query_rel_emb, left_W, left_b, right_W, right_b, center_W, center_b,
           max_edges):
    num_nodes = memorized_embedding.shape[0]
    seg = edges[:, 6]
    dst = edges[:, 7]

    hvi = memorized_embedding[seg]
    hvj = memorized_embedding[dst]

    logits = _dense_logits(
        hvi.reshape(B, E_PER, D), hvj.reshape(B, E_PER, D),
        rel_emb.reshape(B, E_PER, D),
        query_src_ts_emb.reshape(B, 1, D), query_rel_emb.reshape(B, 1, D),
        left_W, right_W, center_W, left_b, right_b, center_b).reshape(-1)

    m = jax.ops.segment_max(logits, seg, num_segments=num_nodes)
    ex = jnp.exp(logits - m[seg])
    s = jax.ops.segment_sum(ex, seg, num_segments=num_nodes)
    target_att = (ex / s[seg]) * node_score[seg]

    pruned, oidx = _topk(target_att.reshape(B, E_PER), max_edges)
    pruned_att_flat = pruned.reshape(-1)
    orig_indices = oidx.reshape(-1)

    pruned_dst = jnp.take(dst, orig_indices, axis=0)
    updated_node_score = jnp.zeros((num_nodes,), jnp.float32).at[pruned_dst].add(pruned_att_flat)
    return (updated_node_score, pruned_att_flat, orig_indices)


# speedup vs baseline: 6.9191x; 6.4598x over previous
import functools

import jax
import jax.numpy as jnp
from jax import lax
from jax.experimental import pallas as pl
from jax.experimental.pallas import tpu as pltpu
from jax.experimental.pallas import tpu_sc as plsc

N = 160000
B = 64
E_PER = N // B
D = 128
K = 64
NN = 10000

NW = 32
PER_W = N // NW
GROWS = 40
NT = 16
PER_T = 10112
N_SM = NT * PER_T
TSTEPS = PER_T // 16
TROWS = PER_T // 128
NB = 10240
BSL = NB // NT

_f32 = jnp.float32
_i32 = jnp.int32


def _gather_body(idxp, mem, gl, gr, idx, rows_a, rows_b, table, sem_a, sem_b):
    wid = lax.axis_index("s") * 2 + lax.axis_index("c")
    base = wid * PER_W
    pltpu.sync_copy(idxp.at[wid], idx)

    @pl.when(lax.axis_index("s") == 0)
    def _stage():
        pltpu.sync_copy(mem, table)

    plsc.subcore_barrier()

    def issue(c, rows, sem):
        pltpu.async_copy(table.at[idx.at[c]], rows, sem)

    def wait(rows, sem):
        pltpu.make_async_copy(mem.at[pl.ds(0, 128)], rows, sem).wait()

    def write(c, rows):
        @pl.when(c < GROWS - 1)
        def _():
            pltpu.sync_copy(rows, gl.at[pl.ds(base + c * 128, 128)])

        @pl.when(c == GROWS - 1)
        def _():
            pltpu.sync_copy(rows.at[pl.ds(0, 8)],
                            gl.at[pl.ds(base + (GROWS - 1) * 128, 8)])

        @pl.when(jnp.logical_and(c >= GROWS, c < 2 * GROWS - 1))
        def _():
            pltpu.sync_copy(rows, gr.at[pl.ds(base + (c - GROWS) * 128, 128)])

        @pl.when(c == 2 * GROWS - 1)
        def _():
            pltpu.sync_copy(rows.at[pl.ds(0, 8)],
                            gr.at[pl.ds(base + (GROWS - 1) * 128, 8)])

    issue(0, rows_a, sem_a)

    def body(c, _):
        @pl.when(c % 2 == 0)
        def _():
            @pl.when(c + 1 < 2 * GROWS)
            def _():
                issue(c + 1, rows_b, sem_b)
            wait(rows_a, sem_a)
            write(c, rows_a)

        @pl.when(c % 2 == 1)
        def _():
            @pl.when(c + 1 < 2 * GROWS)
            def _():
                issue(c + 1, rows_a, sem_a)
            wait(rows_b, sem_b)
            write(c, rows_b)
        return 0

    lax.fori_loop(0, 2 * GROWS, body, 0)


def _sc_gather(idxp, mem):
    mesh = plsc.VectorSubcoreMesh(core_axis_name="c", subcore_axis_name="s")
    kfn = functools.partial(
        pl.kernel,
        out_type=[jax.ShapeDtypeStruct((N, D), _f32),
                  jax.ShapeDtypeStruct((N, D), _f32)],
        mesh=mesh,
        compiler_params=pltpu.CompilerParams(needs_layout_passes=False),
        scratch_types=[pltpu.VMEM((2 * GROWS, 128), _i32),
                       pltpu.VMEM((128, D), _f32),
                       pltpu.VMEM((128, D), _f32),
                       pltpu.VMEM_SHARED((NN, D), _f32),
                       pltpu.SemaphoreType.DMA,
                       pltpu.SemaphoreType.DMA],
    )(_gather_body)
    return kfn(idxp, mem)


def _dense_body(gl_ref, gr_ref, rel_ref, qs_ref, qr_ref, lw_ref, rw_ref,
                cw_ref, lb_ref, rb_ref, cb_ref, out_ref):
    g = gl_ref[0]
    h = gr_ref[0]
    r = rel_ref[0]
    qs = qs_ref[0]
    qr = qr_ref[0]

    dn = (((1,), (1,)), ((), ()))

    def dot(a, b):
        return lax.dot_general(a, b, dn, preferred_element_type=_f32)

    qcat = jnp.concatenate([qs, qr], axis=1)
    ql = dot(qcat, lw_ref[:, 256:512]) + lb_ref[...]
    qright = dot(qcat, rw_ref[:, 256:512]) + rb_ref[...]

    x = jnp.concatenate([g, r], axis=1)
    y = jnp.concatenate([h, r], axis=1)
    lp = dot(x, lw_ref[:, 0:256]) + ql
    rp = dot(y, rw_ref[:, 0:256]) + qright
    left = jnp.where(lp > 0, lp, 0.01 * lp)
    right = jnp.where(rp > 0, rp, 0.01 * rp)
    center = dot(right, cw_ref[...]) + cb_ref[...]
    out_ref[0, 0, :] = jnp.sum(left * center, axis=1)


def _dense_logits(gl, gr, rel, qse, qre, lw, rw, cw, lb, rb, cb):
    full = lambda shape: pl.BlockSpec(shape, lambda q: (0,) * len(shape))
    per_q3 = pl.BlockSpec((1, E_PER, D), lambda q: (q, 0, 0))
    per_q1 = pl.BlockSpec((1, 1, D), lambda q: (q, 0, 0))
    out = pl.pallas_call(
        _dense_body,
        grid=(B,),
        in_specs=[per_q3, per_q3, per_q3, per_q1, per_q1,
                  full((2 * D, 4 * D)), full((2 * D, 4 * D)),
                  full((2 * D, 2 * D)),
                  full((1, 2 * D)), full((1, 2 * D)), full((1, 2 * D))],
        out_specs=pl.BlockSpec((1, 1, E_PER), lambda q: (q, 0, 0)),
        out_shape=jax.ShapeDtypeStruct((B, 1, E_PER), _f32),
    )(gl, gr, rel, qse, qre, lw, rw, cw, lb.reshape(1, -1),
      rb.reshape(1, -1), cb.reshape(1, -1))
    return out.reshape(N)


def _softmax_body(lg_hbm, segf_hbm, seg3_hbm, ns_hbm, out_hbm,
                  bins, sns, nsv, segf, seg2, lgx, tmp, acc, stage, ksh, ssh):
    tid = lax.axis_index("s")
    cid = lax.axis_index("c")
    on0 = cid == 0
    base = tid * PER_T

    z16f = jnp.zeros((16,), _f32)
    neg16 = jnp.full((16,), -1e30, _f32)
    z16i = jnp.zeros((16,), _i32)

    @pl.when(on0)
    def _phase1():
        pltpu.sync_copy(lg_hbm.at[pl.ds(base, PER_T)], lgx)
        pltpu.sync_copy(segf_hbm.at[pl.ds(base, PER_T)], segf)
        pltpu.sync_copy(seg3_hbm.at[tid], seg2)

        def initb(i, _):
            bins[pl.ds(i * 16, 16)] = neg16
            return 0
        lax.fori_loop(0, NB // 16, initb, 0)

        def spass(_c):
            def sbody(j, viol):
                sl = pl.ds(j * 16, 16)
                idx = segf[sl]
                v = lgx[sl]
                cur = plsc.load_gather(bins, [idx])
                upd = v > cur
                plsc.store_scatter(bins, [idx], jnp.maximum(cur, v), mask=upd)
                return viol + jnp.where(upd, 1, 0)
            vv = lax.fori_loop(0, TSTEPS, sbody, z16i)
            return jnp.sum(vv)

        c0 = spass(0)
        lax.while_loop(lambda c: c > 0, spass, c0)

        def stg(i, _):
            pltpu.sync_copy(bins.at[pl.ds(i * BSL, BSL)], stage.at[tid, i])
            return 0
        lax.fori_loop(0, NT, stg, 0)

    plsc.subcore_barrier()

    @pl.when(on0)
    def _phase2():
        pltpu.sync_copy(stage.at[0, tid], acc)

        def cbody(i, _):
            pltpu.sync_copy(stage.at[i, tid], tmp)

            def mx(j, _2):
                sl = pl.ds(j * 16, 16)
                acc[sl] = jnp.maximum(acc[sl], tmp[sl])
                return 0
            lax.fori_loop(0, BSL // 16, mx, 0)
            return 0
        lax.fori_loop(1, NT, cbody, 0)
        pltpu.sync_copy(acc, ksh.at[tid])

        @pl.when(tid == 0)
        def _zero():
            def zb(i, _):
                sns[pl.ds(i * 16, 16)] = z16f
                return 0
            lax.fori_loop(0, NB // 16, zb, 0)
            pltpu.sync_copy(sns, ssh)

    plsc.subcore_barrier()

    @pl.when(on0)
    def _phase3():
        def ld(i, _):
            pltpu.sync_copy(ksh.at[i], bins.at[pl.ds(i * BSL, BSL)])
            return 0
        lax.fori_loop(0, NT, ld, 0)

        def ebody(j, _):
            sl = pl.ds(j * 16, 16)
            kk = plsc.load_gather(bins, [segf[sl]])
            lgx[sl] = jnp.exp(lgx[sl] - kk)
            return 0
        lax.fori_loop(0, TSTEPS, ebody, 0)

        def abody(c, _):
            pltpu.sync_copy(lgx.at[pl.ds(c * 128, 128)],
                            ssh.at[seg2.at[c]], add=True)
            return 0
        lax.fori_loop(0, TROWS, abody, 0)

    plsc.subcore_barrier()

    @pl.when(on0)
    def _phase4():
        pltpu.sync_copy(ssh, sns)
        pltpu.sync_copy(ns_hbm, nsv.at[pl.ds(0, NN)])

        def tbody(j, _):
            sl = pl.ds(j * 16, 16)
            idx = segf[sl]
            s = plsc.load_gather(sns, [idx])
            ns = plsc.load_gather(nsv, [idx])
            lgx[sl] = lgx[sl] * ns / s
            return 0
        lax.fori_loop(0, TSTEPS, tbody, 0)
        pltpu.sync_copy(lgx, out_hbm.at[pl.ds(base, PER_T)])


def _sc_softmax(lg_pad, segf_pad, seg3_pad, node_score):
    mesh = plsc.VectorSubcoreMesh(core_axis_name="c", subcore_axis_name="s")
    kfn = functools.partial(
        pl.kernel,
        out_type=jax.ShapeDtypeStruct((N_SM,), _f32),
        mesh=mesh,
        compiler_params=pltpu.CompilerParams(needs_layout_passes=False),
        scratch_types=[pltpu.VMEM((NB,), _f32),
                       pltpu.VMEM((NB,), _f32),
                       pltpu.VMEM((NB,), _f32),
                       pltpu.VMEM((PER_T,), _i32),
                       pltpu.VMEM((TROWS, 128), _i32),
                       pltpu.VMEM((PER_T,), _f32),
                       pltpu.VMEM((BSL,), _f32),
                       pltpu.VMEM((BSL,), _f32),
                       pltpu.VMEM_SHARED((NT, NT, BSL), _f32),
                       pltpu.VMEM_SHARED((NT, BSL), _f32),
                       pltpu.VMEM_SHARED((NB,), _f32)],
    )(_softmax_body)
    return kfn(lg_pad, segf_pad, seg3_pad, node_score)


def _topk_body(me_ref, ta_ref, vals_ref, idx_ref, work_ref):
    work_ref[...] = ta_ref[...]
    col = lax.broadcasted_iota(_i32, (B, E_PER), 1)
    kcol = lax.broadcasted_iota(_i32, (B, K), 1)
    krow = lax.broadcasted_iota(_i32, (B, K), 0)

    def step(i, carry):
        vals, idxs = carry
        w = work_ref[...]
        m = jnp.max(w, axis=1)
        eq = w == m[:, None]
        amx = jnp.min(jnp.where(eq, col, E_PER), axis=1)
        vals = jnp.where(kcol == i, m[:, None], vals)
        idxs = jnp.where(kcol == i, amx[:, None], idxs)
        work_ref[...] = jnp.where(col == amx[:, None], jnp.float32(-1.0), w)
        return vals, idxs

    vals0 = jnp.zeros((B, K), _f32)
    idxs0 = jnp.zeros((B, K), _i32)
    vals, idxs = lax.fori_loop(0, K, step, (vals0, idxs0))
    me = me_ref[0]
    vals_ref[...] = jnp.where(kcol < me, vals, jnp.float32(0.0))
    idx_ref[...] = idxs + krow * E_PER


def _topk(ta, max_edges):
    me = jnp.asarray(max_edges, _i32).reshape(1)
    return pl.pallas_call(
        _topk_body,
        in_specs=[pl.BlockSpec(memory_space=pltpu.SMEM),
                  pl.BlockSpec((B, E_PER), lambda: (0, 0))],
        out_specs=[pl.BlockSpec((B, K), lambda: (0, 0)),
                   pl.BlockSpec((B, K), lambda: (0, 0))],
        out_shape=[jax.ShapeDtypeStruct((B, K), _f32),
                   jax.ShapeDtypeStruct((B, K), _i32)],
        scratch_shapes=[pltpu.VMEM((B, E_PER), _f32)],
    )(me, ta)


def _scatter_body(oidx_hbm, pr_hbm, dst_hbm, out_hbm,
                  oix, prv, dvals, zb, ssh, sem):
    wid = lax.axis_index("s") * 2 + lax.axis_index("c")

    @pl.when(wid == 0)
    def _run():
        pltpu.sync_copy(oidx_hbm, oix)
        pltpu.sync_copy(pr_hbm, prv)

        z16f = jnp.zeros((16,), _f32)

        def zbody(i, _):
            zb[pl.ds(i * 16, 16)] = z16f
            return 0
        lax.fori_loop(0, NB // 16, zbody, 0)
        pltpu.sync_copy(zb, ssh)

        def body(c, _):
            pltpu.async_copy(dst_hbm.at[oix.at[c]], dvals, sem).wait()
            pltpu.sync_copy(prv.at[c], ssh.at[dvals], add=True)
            return 0
        lax.fori_loop(0, (B * K) // 128, body, 0)
        pltpu.sync_copy(ssh, zb)
        pltpu.sync_copy(zb.at[pl.ds(0, NN)], out_hbm)


def _sc_scatter(oidx2d, pruned2d, dst):
    mesh = plsc.VectorSubcoreMesh(core_axis_name="c", subcore_axis_name="s")
    kfn = functools.partial(
        pl.kernel,
        out_type=jax.ShapeDtypeStruct((NN,), _f32),
        mesh=mesh,
        compiler_params=pltpu.CompilerParams(needs_layout_passes=False),
        scratch_types=[pltpu.VMEM(((B * K) // 128, 128), _i32),
                       pltpu.VMEM(((B * K) // 128, 128), _f32),
                       pltpu.VMEM((128,), _i32),
                       pltpu.VMEM((NB,), _f32),
                       pltpu.VMEM_SHARED((NB,), _f32),
                       pltpu.SemaphoreType.DMA],
    )(_scatter_body)
    return kfn(oidx2d, pruned2d, dst)


def kernel(edges, node_score, memorized_embedding, rel_emb, query_src_ts_emb,
           query_rel_emb, left_W, left_b, right_W, right_b, center_W, center_b,
           max_edges):
    seg = edges[:, 6]
    dst = edges[:, 7]

    segp = jnp.pad(seg.reshape(NW, PER_W), ((0, 0), (0, GROWS * 128 - PER_W))
                   ).reshape(NW, GROWS, 128)
    dstp = jnp.pad(dst.reshape(NW, PER_W), ((0, 0), (0, GROWS * 128 - PER_W))
                   ).reshape(NW, GROWS, 128)
    idxp = jnp.concatenate([segp, dstp], axis=1)
    gl, gr = _sc_gather(idxp, memorized_embedding)

    logits = _dense_logits(
        gl.reshape(B, E_PER, D), gr.reshape(B, E_PER, D),
        rel_emb.reshape(B, E_PER, D),
        query_src_ts_emb.reshape(B, 1, D), query_rel_emb.reshape(B, 1, D),
        left_W, right_W, center_W, left_b, right_b, center_b)

    lg_pad = jnp.concatenate([logits, jnp.full((N_SM - N,), -1e30, _f32)])
    seg_pad = jnp.concatenate([seg, jnp.full((N_SM - N,), NB - 1, _i32)])
    target = _sc_softmax(lg_pad, seg_pad, seg_pad.reshape(NT, TROWS, 128),
                         node_score)

    pruned, oidx = _topk(target[:N].reshape(B, E_PER), max_edges)
    pruned_att_flat = pruned.reshape(-1)
    orig_indices = oidx.reshape(-1)

    updated = _sc_scatter(oidx.reshape((B * K) // 128, 128),
                          pruned.reshape((B * K) // 128, 128), dst)
    return (updated, pruned_att_flat, orig_indices)
